# baseline (speedup 1.0000x reference)
import functools

import jax
import jax.numpy as jnp
from jax import lax
from jax.experimental import pallas as pl
from jax.experimental.pallas import tpu as pltpu
from jax.experimental.pallas import tpu_sc as plsc

_INV_TEMP = float(1.0 / 0.07)
_INV_TEMP_LOG2E = float((1.0 / 0.07) * 1.4426950408889634)
_N_CLASSES = 100000
_FEAT = 640
_NPROTO = 50000
_BATCH = 2048
_BLK = 5000
_GRID = _NPROTO // _BLK


_NTILES = 16
_CLS_PER_TILE = _N_CLASSES // _NTILES
_TSLICE = _CLS_PER_TILE + 6
_TBL = _NTILES * _TSLICE
_LABPAD = 51200
_KCHUNK = _LABPAD // _NTILES
_L1CHUNK = 2000
_BCHUNK = _BATCH // _NTILES


def _sc_body(labpad_hbm, by_hbm, proto_hbm,
             winner_hbm, widx_hbm, grows_hbm, table_hbm,
             lab2_v, tfull_v, win_v, by_v, widx_v,
             rows_a, rows_b, sem_a, sem_b):
    cid = lax.axis_index("c")
    sid = lax.axis_index("s")
    _TSOFF = 50016

    @pl.when(cid == 0)
    def _core0():
        lo = sid * _CLS_PER_TILE
        iota = lax.iota(jnp.int32, 16)
        neg1 = jnp.full((16,), -1, jnp.int32)
        cls_per_tile = jnp.full((16,), _CLS_PER_TILE, jnp.int32)

        cp_lab = pltpu.async_copy(labpad_hbm.at[pl.ds(0, _NPROTO)],
                                  tfull_v.at[pl.ds(0, _NPROTO)], sem_a)

        def _init(i, c):
            tfull_v[pl.ds(_TSOFF + i * 16, 16)] = neg1
            return c
        lax.fori_loop(0, _TSLICE // 16, _init, 0)
        cp_lab.wait()
        tslice_v = tfull_v.at[pl.ds(_TSOFF, _TSLICE)]

        def _vec5(i, carry):
            base = i * 80
            for u in range(5):
                lab = tfull_v[pl.ds(base + u * 16, 16)]
                kvec = iota + (base + u * 16)
                m = (lab >= lo) & (lab < lo + _CLS_PER_TILE)
                idx = jnp.where(m, lab - lo, 0)
                plsc.store_scatter(tslice_v, [idx], kvec, mask=m)
            return carry
        with jax.named_scope("p1_scan"):
            lax.fori_loop(0, _NPROTO // 80, _vec5, 0)

        with jax.named_scope("p1_flush_barrier"):
            pltpu.sync_copy(tslice_v,
                            table_hbm.at[pl.ds(sid * _TSLICE, _TSLICE)])
            plsc.subcore_barrier()

        with jax.named_scope("p2_table_bcast"):
            cp_l2 = pltpu.async_copy(
                labpad_hbm.at[pl.ds(sid * _KCHUNK, _KCHUNK)], lab2_v, sem_b)
            pltpu.sync_copy(table_hbm, tfull_v)
            cp_l2.wait()

        k0 = sid * _KCHUNK

        def _win5(i, c):
            base = i * 80
            for u in range(5):
                off = base + u * 16
                lab = lab2_v[pl.ds(off, 16)]
                gidx = lab + lax.div(lab, cls_per_tile) * (_TSLICE - _CLS_PER_TILE)
                g = plsc.load_gather(tfull_v, [gidx])
                kvec = iota + (k0 + off)
                win_v[pl.ds(off, 16)] = jnp.where(g == kvec,
                                                  jnp.float32(1.0),
                                                  jnp.float32(0.0))
            return c
        with jax.named_scope("p2a_winner"):
            lax.fori_loop(0, _KCHUNK // 80, _win5, 0)
            pltpu.sync_copy(win_v, winner_hbm.at[pl.ds(k0, _KCHUNK)])

        b0 = sid * _BCHUNK
        pltpu.sync_copy(by_hbm.at[pl.ds(b0, _BCHUNK)], by_v)
        for i in range(_BCHUNK // 16):
            y = by_v[pl.ds(i * 16, 16)]
            gidx = y + lax.div(y, cls_per_tile) * (_TSLICE - _CLS_PER_TILE)
            widx_v[pl.ds(i * 16, 16)] = plsc.load_gather(tfull_v, [gidx])
        pltpu.sync_copy(widx_v, widx_hbm.at[pl.ds(b0, _BCHUNK)])

        bufs = (rows_a, rows_b)
        sems = (sem_a, sem_b)
        nb = _BCHUNK // 16
        cps = []
        for i in range(nb):
            ridx = jnp.maximum(widx_v[pl.ds(i * 16, 16)], 0)
            cps.append(pltpu.async_copy(proto_hbm.at[ridx], bufs[i % 2],
                                        sems[i % 2]))
            if i >= 1:
                cps[i - 1].wait()
                pltpu.sync_copy(bufs[(i - 1) % 2],
                                grows_hbm.at[pl.ds(b0 + (i - 1) * 16, 16)])
        cps[nb - 1].wait()
        pltpu.sync_copy(bufs[(nb - 1) % 2],
                        grows_hbm.at[pl.ds(b0 + (nb - 1) * 16, 16)])


_sc_call = functools.partial(
    pl.kernel,
    mesh=plsc.VectorSubcoreMesh(core_axis_name="c", subcore_axis_name="s"),
    compiler_params=pltpu.CompilerParams(needs_layout_passes=False),
    out_type=(
        jax.ShapeDtypeStruct((_LABPAD,), jnp.float32),
        jax.ShapeDtypeStruct((_BATCH,), jnp.int32),
        jax.ShapeDtypeStruct((_BATCH, _FEAT), jnp.float32),
        jax.ShapeDtypeStruct((_TBL,), jnp.int32),
    ),
    scratch_types=[
        pltpu.VMEM((_KCHUNK,), jnp.int32),
        pltpu.VMEM((_TBL,), jnp.int32),
        pltpu.VMEM((_KCHUNK,), jnp.float32),
        pltpu.VMEM((_BCHUNK,), jnp.int32),
        pltpu.VMEM((_BCHUNK,), jnp.int32),
        pltpu.VMEM((16, _FEAT), jnp.float32),
        pltpu.VMEM((16, _FEAT), jnp.float32),
        pltpu.SemaphoreType.DMA,
        pltpu.SemaphoreType.DMA,
    ],
)(_sc_body)


def _loss_body(bx_ref, w_ref, proto_ref, win_ref, grows_ref, mask_ref,
               out_ref, z_ref, den_ref):
    j = pl.program_id(0)

    @pl.when(j == 0)
    def _init():
        zz = jnp.dot(bx_ref[...], w_ref[...], preferred_element_type=jnp.float32)
        n = jnp.sqrt(jnp.sum(zz * zz, axis=1, keepdims=True))
        z_ref[...] = (zz / jnp.maximum(n, 1e-12)) * _INV_TEMP_LOG2E
        den_ref[...] = jnp.zeros_like(den_ref)

    p = proto_ref[...]
    pn = jnp.sqrt(jnp.sum(p * p, axis=1, keepdims=True))
    p = p / jnp.maximum(pn, 1e-12)
    s = lax.dot_general(z_ref[...], p, (((1,), (1,)), ((), ())),
                        preferred_element_type=jnp.float32)
    e = jnp.exp2(s) * win_ref[0, 0, :][None, :]
    den_ref[...] += jnp.sum(e, axis=1, keepdims=True)

    @pl.when(j == _GRID - 1)
    def _fin():
        g = grows_ref[...]
        gn = jnp.sqrt(jnp.sum(g * g, axis=1, keepdims=True))
        dot = jnp.sum(z_ref[...] * (g / jnp.maximum(gn, 1e-12)), axis=1,
                      keepdims=True)
        m = mask_ref[...]
        ln2 = jnp.float32(0.6931471805599453)
        lp = m * (dot * ln2 - jnp.log(den_ref[...]))
        out_ref[...] = (-jnp.sum(lp) / jnp.sum(m))[None, None]


def _build_loss_call(interpret=False):
    return pl.pallas_call(
        _loss_body,
        interpret=interpret,
        grid=(_GRID,),
        in_specs=[
            pl.BlockSpec((_BATCH, _FEAT), lambda j: (0, 0)),
            pl.BlockSpec((_FEAT, _FEAT), lambda j: (0, 0)),
            pl.BlockSpec((_BLK, _FEAT), lambda j: (j, 0)),
            pl.BlockSpec((1, 1, _BLK), lambda j: (j, 0, 0)),
            pl.BlockSpec((_BATCH, _FEAT), lambda j: (0, 0)),
            pl.BlockSpec((_BATCH, 1), lambda j: (0, 0)),
        ],
        out_specs=pl.BlockSpec((1, 1), lambda j: (0, 0)),
        out_shape=jax.ShapeDtypeStruct((1, 1), jnp.float32),
        scratch_shapes=[
            pltpu.VMEM((_BATCH, _FEAT), jnp.float32),
            pltpu.VMEM((_BATCH, 1), jnp.float32),
        ],
    )


_loss_call = _build_loss_call()


def kernel(prototype, class_label, batch_x, batch_y, W):
    labpad = jnp.pad(class_label, (0, _LABPAD - _NPROTO))
    winner, widx, grows, _ = _sc_call(labpad, batch_y, prototype)
    winner_f = winner[:_NPROTO].reshape(_GRID, 1, _BLK)
    mask_f = (widx >= 0).astype(jnp.float32).reshape(_BATCH, 1)
    out = _loss_call(batch_x, W, prototype, winner_f, grows, mask_f)
    return out.reshape(())


# speedup vs baseline: 1.0235x; 1.0235x over previous
import functools

import jax
import jax.numpy as jnp
from jax import lax
from jax.experimental import pallas as pl
from jax.experimental.pallas import tpu as pltpu
from jax.experimental.pallas import tpu_sc as plsc

_INV_TEMP = float(1.0 / 0.07)
_INV_TEMP_LOG2E = float((1.0 / 0.07) * 1.4426950408889634)
_N_CLASSES = 100000
_FEAT = 640
_NPROTO = 50000
_BATCH = 2048
_BLK = 5000
_GRID = _NPROTO // _BLK


_NTILES = 16
_CLS_PER_TILE = _N_CLASSES // _NTILES
_TSLICE = _CLS_PER_TILE + 6
_TBL = _NTILES * _TSLICE
_LABPAD = 51200
_KCHUNK = _LABPAD // _NTILES
_L1CHUNK = 2000
_BCHUNK = _BATCH // _NTILES


def _sc_body(labpad_hbm, by_hbm, proto_hbm,
             winner_hbm, widx_hbm, table_hbm,
             lab2_v, tfull_v, win_v, by_v, widx_v, sem_a, sem_b):
    cid = lax.axis_index("c")
    sid = lax.axis_index("s")
    _TSOFF = 50016

    @pl.when(cid == 0)
    def _core0():
        lo = sid * _CLS_PER_TILE
        iota = lax.iota(jnp.int32, 16)
        neg1 = jnp.full((16,), -1, jnp.int32)
        cls_per_tile = jnp.full((16,), _CLS_PER_TILE, jnp.int32)

        cp_lab = pltpu.async_copy(labpad_hbm.at[pl.ds(0, _NPROTO)],
                                  tfull_v.at[pl.ds(0, _NPROTO)], sem_a)

        def _init(i, c):
            tfull_v[pl.ds(_TSOFF + i * 16, 16)] = neg1
            return c
        lax.fori_loop(0, _TSLICE // 16, _init, 0)
        cp_lab.wait()
        tslice_v = tfull_v.at[pl.ds(_TSOFF, _TSLICE)]

        def _vec5(i, carry):
            base = i * 80
            for u in range(5):
                lab = tfull_v[pl.ds(base + u * 16, 16)]
                kvec = iota + (base + u * 16)
                m = (lab >= lo) & (lab < lo + _CLS_PER_TILE)
                idx = jnp.where(m, lab - lo, 0)
                plsc.store_scatter(tslice_v, [idx], kvec, mask=m)
            return carry
        with jax.named_scope("p1_scan"):
            lax.fori_loop(0, _NPROTO // 80, _vec5, 0)

        with jax.named_scope("p1_flush_barrier"):
            pltpu.sync_copy(tslice_v,
                            table_hbm.at[pl.ds(sid * _TSLICE, _TSLICE)])
            plsc.subcore_barrier()

        with jax.named_scope("p2_table_bcast"):
            cp_l2 = pltpu.async_copy(
                labpad_hbm.at[pl.ds(sid * _KCHUNK, _KCHUNK)], lab2_v, sem_b)
            pltpu.sync_copy(table_hbm, tfull_v)
            cp_l2.wait()

        k0 = sid * _KCHUNK

        def _win5(i, c):
            base = i * 80
            for u in range(5):
                off = base + u * 16
                lab = lab2_v[pl.ds(off, 16)]
                gidx = lab + lax.div(lab, cls_per_tile) * (_TSLICE - _CLS_PER_TILE)
                g = plsc.load_gather(tfull_v, [gidx])
                kvec = iota + (k0 + off)
                win_v[pl.ds(off, 16)] = jnp.where(g == kvec,
                                                  jnp.float32(1.0),
                                                  jnp.float32(0.0))
            return c
        with jax.named_scope("p2a_winner"):
            lax.fori_loop(0, _KCHUNK // 80, _win5, 0)
            pltpu.sync_copy(win_v, winner_hbm.at[pl.ds(k0, _KCHUNK)])

        b0 = sid * _BCHUNK
        pltpu.sync_copy(by_hbm.at[pl.ds(b0, _BCHUNK)], by_v)
        for i in range(_BCHUNK // 16):
            y = by_v[pl.ds(i * 16, 16)]
            gidx = y + lax.div(y, cls_per_tile) * (_TSLICE - _CLS_PER_TILE)
            widx_v[pl.ds(i * 16, 16)] = plsc.load_gather(tfull_v, [gidx])
        pltpu.sync_copy(widx_v, widx_hbm.at[pl.ds(b0, _BCHUNK)])


_sc_call = functools.partial(
    pl.kernel,
    mesh=plsc.VectorSubcoreMesh(core_axis_name="c", subcore_axis_name="s"),
    compiler_params=pltpu.CompilerParams(needs_layout_passes=False),
    out_type=(
        jax.ShapeDtypeStruct((_LABPAD,), jnp.float32),
        jax.ShapeDtypeStruct((_BATCH,), jnp.int32),
        jax.ShapeDtypeStruct((_TBL,), jnp.int32),
    ),
    scratch_types=[
        pltpu.VMEM((_KCHUNK,), jnp.int32),
        pltpu.VMEM((_TBL,), jnp.int32),
        pltpu.VMEM((_KCHUNK,), jnp.float32),
        pltpu.VMEM((_BCHUNK,), jnp.int32),
        pltpu.VMEM((_BCHUNK,), jnp.int32),
        pltpu.SemaphoreType.DMA,
        pltpu.SemaphoreType.DMA,
    ],
)(_sc_body)


_BCHUNK2 = _BATCH // (2 * _NTILES)


def _sc2_body(widx_hbm, proto_hbm, grows_hbm,
              widx_v, rows_a, rows_b, sem_a, sem_b):
    cid = lax.axis_index("c")
    sid = lax.axis_index("s")
    wid = sid * 2 + cid
    b0 = wid * _BCHUNK2
    pltpu.sync_copy(widx_hbm.at[pl.ds(b0, _BCHUNK2)], widx_v)
    bufs = (rows_a, rows_b)
    sems = (sem_a, sem_b)
    nb = _BCHUNK2 // 16
    cps = []
    for i in range(nb):
        ridx = jnp.maximum(widx_v[pl.ds(i * 16, 16)], 0)
        cps.append(pltpu.async_copy(proto_hbm.at[ridx], bufs[i % 2],
                                    sems[i % 2]))
        if i >= 1:
            cps[i - 1].wait()
            pltpu.sync_copy(bufs[(i - 1) % 2],
                            grows_hbm.at[pl.ds(b0 + (i - 1) * 16, 16)])
    cps[nb - 1].wait()
    pltpu.sync_copy(bufs[(nb - 1) % 2],
                    grows_hbm.at[pl.ds(b0 + (nb - 1) * 16, 16)])


_sc2_call = functools.partial(
    pl.kernel,
    mesh=plsc.VectorSubcoreMesh(core_axis_name="c", subcore_axis_name="s"),
    compiler_params=pltpu.CompilerParams(needs_layout_passes=False),
    out_type=(
        jax.ShapeDtypeStruct((_BATCH, _FEAT), jnp.float32),
    ),
    scratch_types=[
        pltpu.VMEM((_BCHUNK2,), jnp.int32),
        pltpu.VMEM((16, _FEAT), jnp.float32),
        pltpu.VMEM((16, _FEAT), jnp.float32),
        pltpu.SemaphoreType.DMA,
        pltpu.SemaphoreType.DMA,
    ],
)(_sc2_body)


def _loss_body(bx_ref, w_ref, proto_ref, win_ref,
               den_out, z_out, z_ref, den_ref):
    j = pl.program_id(0)

    @pl.when(j == 0)
    def _init():
        zz = jnp.dot(bx_ref[...], w_ref[...], preferred_element_type=jnp.float32)
        n = jnp.sqrt(jnp.sum(zz * zz, axis=1, keepdims=True))
        z_ref[...] = (zz / jnp.maximum(n, 1e-12)) * _INV_TEMP_LOG2E
        den_ref[...] = jnp.zeros_like(den_ref)

    p = proto_ref[...]
    pn = jnp.sqrt(jnp.sum(p * p, axis=1, keepdims=True))
    p = p / jnp.maximum(pn, 1e-12)
    s = lax.dot_general(z_ref[...], p, (((1,), (1,)), ((), ())),
                        preferred_element_type=jnp.float32)
    e = jnp.exp2(s) * win_ref[0, 0, :][None, :]
    den_ref[...] += jnp.sum(e, axis=1, keepdims=True)

    @pl.when(j == _GRID - 1)
    def _fin():
        den_out[...] = den_ref[...]
        z_out[...] = z_ref[...]


def _epi_body(z_ref, den_ref, grows_ref, mask_ref, out_ref):
    g = grows_ref[...]
    gn = jnp.sqrt(jnp.sum(g * g, axis=1, keepdims=True))
    dot = jnp.sum(z_ref[...] * (g / jnp.maximum(gn, 1e-12)), axis=1,
                  keepdims=True)
    m = mask_ref[...]
    ln2 = jnp.float32(0.6931471805599453)
    lp = m * (dot * ln2 - jnp.log(den_ref[...]))
    out_ref[...] = (-jnp.sum(lp) / jnp.sum(m))[None, None]


_epi_call = pl.pallas_call(
    _epi_body,
    out_shape=jax.ShapeDtypeStruct((1, 1), jnp.float32),
)


def _build_loss_call(interpret=False):
    return pl.pallas_call(
        _loss_body,
        interpret=interpret,
        grid=(_GRID,),
        in_specs=[
            pl.BlockSpec((_BATCH, _FEAT), lambda j: (0, 0)),
            pl.BlockSpec((_FEAT, _FEAT), lambda j: (0, 0)),
            pl.BlockSpec((_BLK, _FEAT), lambda j: (j, 0)),
            pl.BlockSpec((1, 1, _BLK), lambda j: (j, 0, 0)),
        ],
        out_specs=[
            pl.BlockSpec((_BATCH, 1), lambda j: (0, 0)),
            pl.BlockSpec((_BATCH, _FEAT), lambda j: (0, 0)),
        ],
        out_shape=[
            jax.ShapeDtypeStruct((_BATCH, 1), jnp.float32),
            jax.ShapeDtypeStruct((_BATCH, _FEAT), jnp.float32),
        ],
        scratch_shapes=[
            pltpu.VMEM((_BATCH, _FEAT), jnp.float32),
            pltpu.VMEM((_BATCH, 1), jnp.float32),
        ],
    )


_loss_call = _build_loss_call()


def kernel(prototype, class_label, batch_x, batch_y, W):
    labpad = jnp.pad(class_label, (0, _LABPAD - _NPROTO))
    winner, widx, _ = _sc_call(labpad, batch_y, prototype)
    winner_f = winner[:_NPROTO].reshape(_GRID, 1, _BLK)
    mask_f = (widx >= 0).astype(jnp.float32).reshape(_BATCH, 1)
    (grows,) = _sc2_call(widx, prototype)
    den, z = _loss_call(batch_x, W, prototype, winner_f)
    out = _epi_call(z, den, grows, mask_f)
    return out.reshape(())
